# baseline (speedup 1.0000x reference)
import jax
import jax.numpy as jnp
from jax import lax
from jax.experimental import pallas as pl
from jax.experimental.pallas import tpu as pltpu
from jax.experimental.pallas import tpu_sc as plsc

_LANES = 16
_DIM = 32
_BATCH = 16384
_NUM_WORKERS = 32
_BPW = _BATCH // _NUM_WORKERS


def _sc_body(uidx_hbm, iidx_hbm, utab_hbm, itab_hbm, out_hbm,
             uidx_v, iidx_v, urows_v, irows_v, out_v, usem, isem):
    wid = lax.axis_index("s") * 2 + lax.axis_index("c")
    base = wid * _BPW

    pltpu.sync_copy(uidx_hbm.at[pl.ds(base, _BPW)], uidx_v)
    pltpu.sync_copy(iidx_hbm.at[pl.ds(base, _BPW)], iidx_v)

    ug = pltpu.async_copy(utab_hbm.at[uidx_v], urows_v, usem)
    ig = pltpu.async_copy(itab_hbm.at[iidx_v], irows_v, isem)
    ug.wait()
    ig.wait()

    lane = lax.iota(jnp.int32, _LANES)

    def block(blk, _):
        rbase = blk * _LANES
        rows = rbase + lane
        acc = jnp.zeros((_LANES,), jnp.float32)
        for d in range(_DIM):
            col = (lane + d) & (_DIM - 1)
            uu = plsc.load_gather(urows_v, [rows, col])
            vv = plsc.load_gather(irows_v, [rows, col])
            acc = acc + uu * vv
        out_v[pl.ds(rbase, _LANES)] = 1.0 / (1.0 + jnp.exp(-acc))
        return 0

    lax.fori_loop(0, _BPW // _LANES, block, 0)

    pltpu.sync_copy(out_v, out_hbm.at[pl.ds(base, _BPW)])


@jax.jit
def _mf_dot(user_idx, item_idx, user_table, item_table):
    mesh = plsc.VectorSubcoreMesh(core_axis_name="c", subcore_axis_name="s")
    return pl.kernel(
        _sc_body,
        out_type=jax.ShapeDtypeStruct((_BATCH,), jnp.float32),
        mesh=mesh,
        compiler_params=pltpu.CompilerParams(needs_layout_passes=False, use_tc_tiling_on_sc=False),
        scratch_types=[
            pltpu.VMEM((_BPW,), jnp.int32),
            pltpu.VMEM((_BPW,), jnp.int32),
            pltpu.VMEM((_BPW, _DIM), jnp.float32),
            pltpu.VMEM((_BPW, _DIM), jnp.float32),
            pltpu.VMEM((_BPW,), jnp.float32),
            pltpu.SemaphoreType.DMA,
            pltpu.SemaphoreType.DMA,
        ],
    )(user_idx, item_idx, user_table, item_table)


def kernel(inputs, user_table, item_table):
    user_idx = jnp.ravel(inputs[:, 0])
    item_idx = jnp.ravel(inputs[:, 1])
    x = _mf_dot(user_idx, item_idx, user_table, item_table)
    return x.reshape(_BATCH, 1)


# speedup vs baseline: 3.5095x; 3.5095x over previous
import jax
import jax.numpy as jnp
from jax import lax
from jax.experimental import pallas as pl
from jax.experimental.pallas import tpu as pltpu
from jax.experimental.pallas import tpu_sc as plsc

_LANES = 16
_DIM = 32
_GR = 128
_BATCH = 16384
_NUM_WORKERS = 32
_BPW = _BATCH // _NUM_WORKERS
_CHUNK = 8


def _sc_body(uidx_hbm, iidx_hbm, utab_hbm, itab_hbm, out_hbm,
             uidx_v, iidx_v, ubuf, vbuf, out_v, usem, isem):
    wid = lax.axis_index("s") * 2 + lax.axis_index("c")
    base = wid * _BPW

    pltpu.sync_copy(uidx_hbm.at[pl.ds(base, _BPW)], uidx_v)
    pltpu.sync_copy(iidx_hbm.at[pl.ds(base, _BPW)], iidx_v)

    lane = lax.iota(jnp.int32, _LANES)
    low = lane < _CHUNK

    def issue(gvec, tab, buf, sem, lo):
        for t in range(_CHUNK):
            gb = pl.multiple_of(gvec[lo + t], _GR)
            pltpu.async_copy(tab.at[:, pl.ds(gb, _GR)], buf.at[t], sem)

    def drain(tab, buf, sem):
        for t in range(_CHUNK):
            pltpu.make_async_copy(tab.at[:, pl.ds(0, _GR)],
                                  buf.at[t], sem).wait()

    def half_dot(i0, lo, slotv, khalf):
        iref = jnp.full((_LANES,), i0 + lo, jnp.int32) + slotv
        ju = plsc.load_gather(uidx_v, [iref]) & (_GR - 1)
        jv = plsc.load_gather(iidx_v, [iref]) & (_GR - 1)
        acc = jnp.zeros((_LANES,), jnp.float32)
        for kk in range(_LANES):
            kvec = khalf + kk
            uu = plsc.load_gather(ubuf, [slotv, kvec, ju])
            vv = plsc.load_gather(vbuf, [slotv, kvec, jv])
            acc = acc + uu * vv
        return acc + lax.rev(acc, (0,))

    def group(it, _):
        i0 = it * _LANES
        uvec = uidx_v[pl.ds(i0, _LANES)]
        ivec = iidx_v[pl.ds(i0, _LANES)]
        ug = (uvec >> 7) << 7
        ig = (ivec >> 7) << 7

        issue(ug, utab_hbm, ubuf, usem, 0)
        issue(ig, itab_hbm, vbuf, isem, 0)
        drain(utab_hbm, ubuf, usem)
        drain(itab_hbm, vbuf, isem)
        s0 = half_dot(i0, 0, jnp.where(low, lane, 15 - lane),
                      jnp.where(low, 0, _LANES))

        issue(ug, utab_hbm, ubuf, usem, _CHUNK)
        issue(ig, itab_hbm, vbuf, isem, _CHUNK)
        drain(utab_hbm, ubuf, usem)
        drain(itab_hbm, vbuf, isem)
        s1 = half_dot(i0, _CHUNK, jnp.where(low, 7 - lane, lane - 8),
                      jnp.where(low, _LANES, 0))

        x = jnp.where(low, s0, s1)
        out_v[pl.ds(i0, _LANES)] = 1.0 / (1.0 + jnp.exp(-x))
        return 0

    lax.fori_loop(0, _BPW // _LANES, group, 0)

    pltpu.sync_copy(out_v, out_hbm.at[pl.ds(base, _BPW)])


@jax.jit
def _mf_dot(user_idx, item_idx, user_table_t, item_table_t):
    mesh = plsc.VectorSubcoreMesh(core_axis_name="c", subcore_axis_name="s")
    return pl.kernel(
        _sc_body,
        out_type=jax.ShapeDtypeStruct((_BATCH,), jnp.float32),
        mesh=mesh,
        compiler_params=pltpu.CompilerParams(
            needs_layout_passes=False, use_tc_tiling_on_sc=True),
        scratch_types=[
            pltpu.VMEM((_BPW,), jnp.int32),
            pltpu.VMEM((_BPW,), jnp.int32),
            pltpu.VMEM((_CHUNK, _DIM, _GR), jnp.float32),
            pltpu.VMEM((_CHUNK, _DIM, _GR), jnp.float32),
            pltpu.VMEM((_BPW,), jnp.float32),
            pltpu.SemaphoreType.DMA,
            pltpu.SemaphoreType.DMA,
        ],
    )(user_idx, item_idx, user_table_t, item_table_t)


def kernel(inputs, user_table, item_table):
    user_idx = jnp.ravel(inputs[:, 0])
    item_idx = jnp.ravel(inputs[:, 1])
    x = _mf_dot(user_idx, item_idx, user_table.T, item_table.T)
    return x.reshape(_BATCH, 1)


# speedup vs baseline: 3.8694x; 1.1026x over previous
import jax
import jax.numpy as jnp
from jax import lax
from jax.experimental import pallas as pl
from jax.experimental.pallas import tpu as pltpu
from jax.experimental.pallas import tpu_sc as plsc

_LANES = 16
_DIM = 32
_GR = 128
_BATCH = 16384
_NUM_WORKERS = 32
_BPW = _BATCH // _NUM_WORKERS
_CHUNK = 4
_NPAIR = _BPW // (2 * _CHUNK)


def _sc_body(uidx_hbm, iidx_hbm, utab_hbm, itab_hbm, out_hbm,
             uidx_v, iidx_v, ubuf, vbuf, tmp_v, out_v,
             usem0, isem0, usem1, isem1):
    wid = lax.axis_index("s") * 2 + lax.axis_index("c")
    base = wid * _BPW

    pltpu.sync_copy(uidx_hbm.at[pl.ds(base, _BPW)],
                    uidx_v.at[pl.ds(0, _BPW)])
    pltpu.sync_copy(iidx_hbm.at[pl.ds(base, _BPW)],
                    iidx_v.at[pl.ds(0, _BPW)])

    lane = lax.iota(jnp.int32, _LANES)
    quarter = lane >> 2
    lowq = lane < 2 * _CHUNK
    slotv = jnp.where(lowq, lane & (_CHUNK - 1), (15 - lane) & (_CHUNK - 1))
    kbase = quarter << 3
    outmask = lane < _CHUNK
    outoff = lane & (_CHUNK - 1)

    def issue(gu, gi, lo, par, usem, isem):
        for t in range(_CHUNK):
            gb = pl.multiple_of(gu[lo + t], _GR)
            pltpu.async_copy(utab_hbm.at[:, pl.ds(gb, _GR)],
                             ubuf.at[par, t], usem)
        for t in range(_CHUNK):
            gb = pl.multiple_of(gi[lo + t], _GR)
            pltpu.async_copy(itab_hbm.at[:, pl.ds(gb, _GR)],
                             vbuf.at[par, t], isem)

    def drain(par, usem, isem):
        for t in range(_CHUNK):
            pltpu.make_async_copy(utab_hbm.at[:, pl.ds(0, _GR)],
                                  ubuf.at[par, t], usem).wait()
            pltpu.make_async_copy(itab_hbm.at[:, pl.ds(0, _GR)],
                                  vbuf.at[par, t], isem).wait()

    def dots4(i0, par):
        iref = jnp.full((_LANES,), i0, jnp.int32) + slotv
        ju = plsc.load_gather(uidx_v, [iref]) & (_GR - 1)
        jv = plsc.load_gather(iidx_v, [iref]) & (_GR - 1)
        parv = jnp.full((_LANES,), par, jnp.int32)
        acc = jnp.zeros((_LANES,), jnp.float32)
        for kk in range(_DIM // 4):
            kvec = kbase + kk
            uu = plsc.load_gather(ubuf, [parv, slotv, kvec, ju])
            vv = plsc.load_gather(vbuf, [parv, slotv, kvec, jv])
            acc = acc + uu * vv
        s = acc + lax.rev(acc, (0,))
        tmp_v[pl.ds(0, _LANES)] = s
        tot = s + plsc.load_gather(tmp_v, [lane ^ 4])
        plsc.store_scatter(out_v, [jnp.full((_LANES,), i0, jnp.int32)
                                   + outoff], tot, mask=outmask)

    uvec0 = uidx_v[pl.ds(0, _LANES)]
    ivec0 = iidx_v[pl.ds(0, _LANES)]
    issue((uvec0 >> 7) << 7, (ivec0 >> 7) << 7, 0, 0, usem0, isem0)

    def pair(j, _):
        i0 = j * 2 * _CHUNK
        uvec = uidx_v[pl.ds(i0, _LANES)]
        ivec = iidx_v[pl.ds(i0, _LANES)]
        gu = (uvec >> 7) << 7
        gi = (ivec >> 7) << 7

        issue(gu, gi, _CHUNK, 1, usem1, isem1)
        drain(0, usem0, isem0)
        dots4(i0, 0)

        @pl.when(j + 1 < _NPAIR)
        def _():
            issue(gu, gi, 2 * _CHUNK, 0, usem0, isem0)

        drain(1, usem1, isem1)
        dots4(i0 + _CHUNK, 1)
        return 0

    lax.fori_loop(0, _NPAIR, pair, 0)

    for i in range(_BPW // _LANES):
        x = out_v[pl.ds(i * _LANES, _LANES)]
        out_v[pl.ds(i * _LANES, _LANES)] = 1.0 / (1.0 + jnp.exp(-x))

    pltpu.sync_copy(out_v, out_hbm.at[pl.ds(base, _BPW)])


@jax.jit
def _mf_dot(user_idx, item_idx, user_table_t, item_table_t):
    mesh = plsc.VectorSubcoreMesh(core_axis_name="c", subcore_axis_name="s")
    return pl.kernel(
        _sc_body,
        out_type=jax.ShapeDtypeStruct((_BATCH,), jnp.float32),
        mesh=mesh,
        compiler_params=pltpu.CompilerParams(
            needs_layout_passes=False, use_tc_tiling_on_sc=True),
        scratch_types=[
            pltpu.VMEM((_BPW + _LANES,), jnp.int32),
            pltpu.VMEM((_BPW + _LANES,), jnp.int32),
            pltpu.VMEM((2, _CHUNK, _DIM, _GR), jnp.float32),
            pltpu.VMEM((2, _CHUNK, _DIM, _GR), jnp.float32),
            pltpu.VMEM((_LANES,), jnp.float32),
            pltpu.VMEM((_BPW,), jnp.float32),
            pltpu.SemaphoreType.DMA,
            pltpu.SemaphoreType.DMA,
            pltpu.SemaphoreType.DMA,
            pltpu.SemaphoreType.DMA,
        ],
    )(user_idx, item_idx, user_table_t, item_table_t)


def kernel(inputs, user_table, item_table):
    user_idx = jnp.ravel(inputs[:, 0])
    item_idx = jnp.ravel(inputs[:, 1])
    x = _mf_dot(user_idx, item_idx, user_table.T, item_table.T)
    return x.reshape(_BATCH, 1)
